# baseline (speedup 1.0000x reference)
import functools

import jax
import jax.numpy as jnp
from jax import lax
from jax.experimental import pallas as pl
from jax.experimental.pallas import tpu as pltpu
from jax.experimental.pallas import tpu_sc as plsc

_VOCAB = 1000
_SEQ = 50
_BATCH = 1024
_NRATINGS = (_SEQ - 1) // 2
_NC = 2
_NS = 16
_NW = _NC * _NS
_RPW = _BATCH // _NW


def _sc_body(x_hbm, z_hbm, out_hbm, row_img, xrow):
    c = lax.axis_index("c")
    s = lax.axis_index("s")
    wid = s * _NC + c
    base = wid * _RPW
    pltpu.sync_copy(z_hbm, row_img)
    lanes = lax.iota(jnp.int32, 16)

    def rbody(j, carry):
        b = base + j
        pltpu.sync_copy(x_hbm.at[b], xrow)
        g1 = plsc.load_gather(xrow, [2 + 2 * lanes])
        m2 = lanes < (_NRATINGS - 16)
        g2 = plsc.load_gather(xrow, [jnp.where(m2, 34 + 2 * lanes, 0)])
        g2 = jnp.where(m2, g2, 0)
        tot = jnp.sum(g1 + g2)
        q = tot // _NRATINGS
        r = tot - q * _NRATINGS
        half = _NRATINGS // 2
        inc = jnp.where((r > half) | ((r == half) & ((q & 1) == 1)), 1, 0)
        avg = q + inc
        rowv = jnp.full((16,), _SEQ - 1, jnp.int32)
        colv = jnp.full((16,), avg, jnp.int32)
        m0 = lanes == 0
        plsc.store_scatter(row_img, [rowv, colv], jnp.full((16,), 1.0, jnp.float32), mask=m0)
        pltpu.sync_copy(row_img, out_hbm.at[b])
        plsc.store_scatter(row_img, [rowv, colv], jnp.zeros((16,), jnp.float32), mask=m0)
        return carry

    lax.fori_loop(0, _RPW, rbody, 0)


@jax.jit
def kernel(x):
    z = jnp.zeros((_SEQ, _VOCAB), jnp.float32)
    mesh = plsc.VectorSubcoreMesh(
        core_axis_name="c", subcore_axis_name="s",
        num_cores=_NC, num_subcores=_NS,
    )
    f = pl.kernel(
        _sc_body,
        out_type=jax.ShapeDtypeStruct((_BATCH, _SEQ, _VOCAB), jnp.float32),
        mesh=mesh,
        scratch_types=[
            pltpu.VMEM((_SEQ, _VOCAB), jnp.float32),
            pltpu.VMEM((_SEQ,), jnp.int32),
        ],
        compiler_params=pltpu.CompilerParams(needs_layout_passes=False),
    )
    return f(x, z)


# speedup vs baseline: 1.0241x; 1.0241x over previous
import jax
import jax.numpy as jnp
from jax import lax
from jax.experimental import pallas as pl
from jax.experimental.pallas import tpu as pltpu
from jax.experimental.pallas import tpu_sc as plsc

_VOCAB = 1000
_SEQ = 50
_BATCH = 1024
_NRATINGS = (_SEQ - 1) // 2
_NC = 2
_NS = 16
_NW = _NC * _NS
_RPW = _BATCH // _NW
_CHUNK = 8


def _sc_body(x_hbm, z_hbm, out_hbm, zimg, planes, xv, sem_a, sem_b):
    c = lax.axis_index("c")
    s = lax.axis_index("s")
    wid = s * _NC + c
    base = wid * _RPW
    pltpu.sync_copy(z_hbm, zimg)
    pltpu.sync_copy(z_hbm.at[pl.ds(0, _RPW), :], planes)
    pltpu.sync_copy(x_hbm.at[pl.ds(base, _RPW)], xv)
    lanes = lax.iota(jnp.int32, 16)
    m2 = lanes < (_NRATINGS - 16)
    idx1 = 2 + 2 * lanes
    idx2 = jnp.where(m2, 2 + 2 * (16 + lanes), 0)

    def avg_body(j, carry):
        g1 = plsc.load_gather(xv, [jnp.full((16,), j, jnp.int32), idx1])
        g2 = plsc.load_gather(xv, [jnp.full((16,), j, jnp.int32), idx2])
        tot = jnp.sum(g1 + jnp.where(m2, g2, 0))
        q = tot // _NRATINGS
        r = tot - q * _NRATINGS
        half = _NRATINGS // 2
        inc = jnp.where((r > half) | ((r == half) & ((q & 1) == 1)), 1, 0)
        avg = q + inc
        plsc.store_scatter(
            planes,
            [jnp.full((16,), j, jnp.int32), jnp.full((16,), avg, jnp.int32)],
            jnp.full((16,), 1.0, jnp.float32),
            mask=lanes == 0,
        )
        return carry

    lax.fori_loop(0, _RPW, avg_body, 0)

    for c0 in range(0, _RPW, _CHUNK):
        def fire_img(j, carry):
            b = base + c0 + j
            pltpu.make_async_copy(zimg, out_hbm.at[b], sem_a).start()
            return carry

        def drain_img(j, carry):
            b = base + c0 + j
            pltpu.make_async_copy(zimg, out_hbm.at[b], sem_a).wait()
            return carry

        def fire_plane(j, carry):
            b = base + c0 + j
            pltpu.make_async_copy(
                planes.at[c0 + j], out_hbm.at[b, _SEQ - 1], sem_b
            ).start()
            return carry

        lax.fori_loop(0, _CHUNK, fire_img, 0)
        lax.fori_loop(0, _CHUNK, drain_img, 0)
        lax.fori_loop(0, _CHUNK, fire_plane, 0)

    def drain_plane(j, carry):
        b = base + j
        pltpu.make_async_copy(
            planes.at[j], out_hbm.at[b, _SEQ - 1], sem_b
        ).wait()
        return carry

    lax.fori_loop(0, _RPW, drain_plane, 0)


@jax.jit
def kernel(x):
    z = jnp.zeros((_SEQ, _VOCAB), jnp.float32)
    mesh = plsc.VectorSubcoreMesh(
        core_axis_name="c", subcore_axis_name="s",
        num_cores=_NC, num_subcores=_NS,
    )
    f = pl.kernel(
        _sc_body,
        out_type=jax.ShapeDtypeStruct((_BATCH, _SEQ, _VOCAB), jnp.float32),
        mesh=mesh,
        scratch_types=[
            pltpu.VMEM((_SEQ, _VOCAB), jnp.float32),
            pltpu.VMEM((_RPW, _VOCAB), jnp.float32),
            pltpu.VMEM((_RPW, _SEQ), jnp.int32),
            pltpu.SemaphoreType.DMA,
            pltpu.SemaphoreType.DMA,
        ],
        compiler_params=pltpu.CompilerParams(needs_layout_passes=False),
    )
    return f(x, z)
